# baseline (speedup 1.0000x reference)
import functools

import jax
import jax.numpy as jnp
from jax import lax
from jax.experimental import pallas as pl
from jax.experimental.pallas import tpu as pltpu
from jax.experimental.pallas import tpu_sc as plsc

BATCH = 1024
CONTEXT = 20
EMBED = 64
IDX_CHUNK = 80
VBK = 16384


def _tc_pair_table(emb_t, vbk=VBK):
    d, v = emb_t.shape
    grid = pl.cdiv(v, vbk)
    half = vbk // 2

    def body(w_ref, o_ref):
        row = lax.broadcasted_iota(jnp.int32, (d, d), 0)
        col = lax.broadcasted_iota(jnp.int32, (d, d), 1)
        eye = (row == col).astype(jnp.float32)
        t = lax.dot_general(
            w_ref[...], eye, (((0,), (0,)), ((), ())),
            precision=lax.Precision.HIGHEST,
            preferred_element_type=jnp.float32,
        )
        o_ref[...] = jnp.concatenate([t[:half], t[half:]], axis=1)

    return pl.pallas_call(
        body,
        grid=(grid,),
        in_specs=[pl.BlockSpec((d, vbk), lambda j: (0, j))],
        out_specs=pl.BlockSpec((half, 2 * d), lambda j: (j, 0)),
        out_shape=jax.ShapeDtypeStruct((grid * half, 2 * d), jnp.float32),
    )(emb_t)


def _sc_gather_mean(idx2d, pair_tab):
    sh_blk = VBK.bit_length() - 1
    sh_half = sh_blk - 1
    mask_half = (VBK // 2) - 1
    info = plsc.get_sparse_core_info()
    nc, ns = info.num_cores, info.num_subcores
    nw = nc * ns
    b_per_w = BATCH // nw
    rows_per_w = b_per_w * CONTEXT
    chunks_per_w = rows_per_w // IDX_CHUNK

    mesh = plsc.VectorSubcoreMesh(core_axis_name="c", subcore_axis_name="s")

    @functools.partial(
        pl.kernel,
        mesh=mesh,
        out_type=jax.ShapeDtypeStruct((BATCH, EMBED), jnp.float32),
        scratch_types=[
            pltpu.VMEM((chunks_per_w, IDX_CHUNK), jnp.int32),
            pltpu.VMEM((chunks_per_w, IDX_CHUNK), jnp.int32),
            pltpu.VMEM((rows_per_w, 2 * EMBED), jnp.float32),
            pltpu.VMEM((b_per_w, EMBED), jnp.float32),
            pltpu.SemaphoreType.DMA,
        ],
        compiler_params=pltpu.CompilerParams(use_tc_tiling_on_sc=False, needs_layout_passes=False),
    )
    def sc_kernel(idx_hbm, tab_hbm, out_hbm,
                  idx_v, off_v, rows_v, acc_v, sem):
        wid = lax.axis_index("s") * nc + lax.axis_index("c")
        base = wid * chunks_per_w
        pltpu.sync_copy(idx_hbm.at[pl.ds(base, chunks_per_w)], idx_v)
        for r in range(chunks_per_w):
            for c in range(0, IDX_CHUNK, 16):
                sl = pl.ds(c, 16)
                v = idx_v[r, sl]
                off_v[r, sl] = ((v >> sh_half) & 1) << 6
                idx_v[r, sl] = ((v >> sh_blk) << sh_half) | (v & mask_half)
        copies = []
        for j in range(chunks_per_w):
            copies.append(pltpu.async_copy(
                tab_hbm.at[idx_v.at[j]],
                rows_v.at[pl.ds(j * IDX_CHUNK, IDX_CHUNK)],
                sem))
        for c in copies:
            c.wait()

        inv = jnp.float32(1.0 / CONTEXT)
        lanes = lax.iota(jnp.int32, 16)

        def body(b, carry):
            row0 = b * CONTEXT
            accs = [jnp.zeros((16,), jnp.float32) for _ in range(4)]
            for t in range(CONTEXT):
                j = row0 + t
                jr = jnp.full((16,), j // IDX_CHUNK, jnp.int32)
                jc = jnp.full((16,), j % IDX_CHUNK, jnp.int32)
                off = plsc.load_gather(off_v, [jr, jc])
                jrow = jnp.full((16,), j, jnp.int32)
                for d in range(4):
                    col = off + (d * 16 + lanes)
                    accs[d] = accs[d] + plsc.load_gather(rows_v, [jrow, col])
            for d in range(4):
                acc_v[b, pl.ds(d * 16, 16)] = accs[d] * inv
            return carry

        lax.fori_loop(0, b_per_w, body, 0)
        pltpu.sync_copy(acc_v, out_hbm.at[pl.ds(wid * b_per_w, b_per_w)])

    return sc_kernel(idx2d, pair_tab)


def _tc_matmul_t(ctx, w_t, linear_b, vb=4096):
    v = w_t.shape[1]
    grid = pl.cdiv(v, vb)

    def mm_body(w_ref, x_ref, b_ref, o_ref):
        o_ref[...] = lax.dot_general(
            w_ref[...], x_ref[...],
            (((0,), (1,)), ((), ())),
            preferred_element_type=jnp.float32,
        ) + b_ref[...][:, None]

    return pl.pallas_call(
        mm_body,
        grid=(grid,),
        in_specs=[
            pl.BlockSpec((EMBED, vb), lambda j: (0, j)),
            pl.BlockSpec((BATCH, EMBED), lambda j: (0, 0)),
            pl.BlockSpec((vb,), lambda j: (j,)),
        ],
        out_specs=pl.BlockSpec((vb, BATCH), lambda j: (j, 0)),
        out_shape=jax.ShapeDtypeStruct((v, BATCH), jnp.float32),
    )(w_t, ctx, linear_b)


def kernel(context_words, embeddings, linear_w, linear_b):
    idx2d = context_words.astype(jnp.int32).reshape(-1, IDX_CHUNK)
    pair_tab = _tc_pair_table(embeddings.T)
    ctx = _sc_gather_mean(idx2d, pair_tab)
    out_t = _tc_matmul_t(ctx, linear_w.T, linear_b)
    return out_t.T


# speedup vs baseline: 1.1233x; 1.1233x over previous
import functools

import jax
import jax.numpy as jnp
from jax import lax
from jax.experimental import pallas as pl
from jax.experimental.pallas import tpu as pltpu
from jax.experimental.pallas import tpu_sc as plsc

BATCH = 1024
CONTEXT = 20
EMBED = 64
IDX_CHUNK = 80
VBK = 8192


def _tc_pair_table(emb_t, vbk=VBK):
    d, v = emb_t.shape
    grid = pl.cdiv(v, vbk)
    half = vbk // 2

    def body(w_ref, o_ref):
        t = jnp.transpose(w_ref[...])
        o_ref[...] = jnp.concatenate([t[:half], t[half:]], axis=1)

    return pl.pallas_call(
        body,
        grid=(grid,),
        in_specs=[pl.BlockSpec((d, vbk), lambda j: (0, j))],
        out_specs=pl.BlockSpec((half, 2 * d), lambda j: (j, 0)),
        out_shape=jax.ShapeDtypeStruct((grid * half, 2 * d), jnp.float32),
    )(emb_t)


def _sc_gather_mean(idx2d, pair_tab):
    sh_blk = VBK.bit_length() - 1
    sh_half = sh_blk - 1
    mask_half = (VBK // 2) - 1
    info = plsc.get_sparse_core_info()
    nc, ns = info.num_cores, info.num_subcores
    nw = nc * ns
    b_per_w = BATCH // nw
    rows_per_w = b_per_w * CONTEXT
    chunks_per_w = rows_per_w // IDX_CHUNK

    mesh = plsc.VectorSubcoreMesh(core_axis_name="c", subcore_axis_name="s")

    @functools.partial(
        pl.kernel,
        mesh=mesh,
        out_type=jax.ShapeDtypeStruct((BATCH, EMBED), jnp.float32),
        scratch_types=[
            pltpu.VMEM((chunks_per_w, IDX_CHUNK), jnp.int32),
            pltpu.VMEM((chunks_per_w, IDX_CHUNK), jnp.int32),
            pltpu.VMEM((rows_per_w, 2 * EMBED), jnp.float32),
            pltpu.VMEM((b_per_w, EMBED), jnp.float32),
            pltpu.SemaphoreType.DMA,
        ],
        compiler_params=pltpu.CompilerParams(use_tc_tiling_on_sc=False, needs_layout_passes=False),
    )
    def sc_kernel(idx_hbm, tab_hbm, out_hbm,
                  idx_v, off_v, rows_v, acc_v, sem):
        wid = lax.axis_index("s") * nc + lax.axis_index("c")
        base = wid * chunks_per_w
        pltpu.sync_copy(idx_hbm.at[pl.ds(base, chunks_per_w)], idx_v)
        for r in range(chunks_per_w):
            for c in range(0, IDX_CHUNK, 16):
                sl = pl.ds(c, 16)
                v = idx_v[r, sl]
                off_v[r, sl] = ((v >> sh_half) & 1) << 6
                idx_v[r, sl] = ((v >> sh_blk) << sh_half) | (v & mask_half)
        copies = []
        for j in range(chunks_per_w):
            copies.append(pltpu.async_copy(
                tab_hbm.at[idx_v.at[j]],
                rows_v.at[pl.ds(j * IDX_CHUNK, IDX_CHUNK)],
                sem))
        for c in copies:
            c.wait()

        inv = jnp.float32(1.0 / CONTEXT)
        lanes = lax.iota(jnp.int32, 16)

        def body(b, carry):
            row0 = b * CONTEXT
            accs = [jnp.zeros((16,), jnp.float32) for _ in range(4)]
            for t in range(CONTEXT):
                j = row0 + t
                jr = jnp.full((16,), j // IDX_CHUNK, jnp.int32)
                jc = jnp.full((16,), j % IDX_CHUNK, jnp.int32)
                off = plsc.load_gather(off_v, [jr, jc])
                jrow = jnp.full((16,), j, jnp.int32)
                for d in range(4):
                    col = off + (d * 16 + lanes)
                    accs[d] = accs[d] + plsc.load_gather(rows_v, [jrow, col])
            for d in range(4):
                acc_v[b, pl.ds(d * 16, 16)] = accs[d] * inv
            return carry

        lax.fori_loop(0, b_per_w, body, 0)
        pltpu.sync_copy(acc_v, out_hbm.at[pl.ds(wid * b_per_w, b_per_w)])

    return sc_kernel(idx2d, pair_tab)


def _tc_matmul_t(ctx, w_t, linear_b, vb=4096):
    v = w_t.shape[1]
    grid = pl.cdiv(v, vb)

    def mm_body(w_ref, x_ref, b_ref, o_ref):
        o_ref[...] = lax.dot_general(
            w_ref[...], x_ref[...],
            (((0,), (1,)), ((), ())),
            preferred_element_type=jnp.float32,
        ) + b_ref[...][:, None]

    return pl.pallas_call(
        mm_body,
        grid=(grid,),
        in_specs=[
            pl.BlockSpec((EMBED, vb), lambda j: (0, j)),
            pl.BlockSpec((BATCH, EMBED), lambda j: (0, 0)),
            pl.BlockSpec((vb,), lambda j: (j,)),
        ],
        out_specs=pl.BlockSpec((vb, BATCH), lambda j: (j, 0)),
        out_shape=jax.ShapeDtypeStruct((v, BATCH), jnp.float32),
    )(w_t, ctx, linear_b)


def kernel(context_words, embeddings, linear_w, linear_b):
    idx2d = context_words.astype(jnp.int32).reshape(-1, IDX_CHUNK)
    pair_tab = _tc_pair_table(embeddings.T)
    ctx = _sc_gather_mean(idx2d, pair_tab)
    out_t = _tc_matmul_t(ctx, linear_w.T, linear_b)
    return out_t.T


# speedup vs baseline: 1.1253x; 1.0019x over previous
import functools

import jax
import jax.numpy as jnp
from jax import lax
from jax.experimental import pallas as pl
from jax.experimental.pallas import tpu as pltpu
from jax.experimental.pallas import tpu_sc as plsc

BATCH = 1024
CONTEXT = 20
EMBED = 64
IDX_CHUNK = 80
VBK = 8192


def _tc_pair_table(emb_t, vbk=VBK):
    d, v = emb_t.shape
    grid = pl.cdiv(v, vbk)
    half = vbk // 2

    def body(w_ref, o_ref):
        t = jnp.transpose(w_ref[...])
        o_ref[:, 0:d] = t[:half]
        o_ref[:, d:2 * d] = t[half:]

    return pl.pallas_call(
        body,
        grid=(grid,),
        in_specs=[pl.BlockSpec((d, vbk), lambda j: (0, j))],
        out_specs=pl.BlockSpec((half, 2 * d), lambda j: (j, 0)),
        out_shape=jax.ShapeDtypeStruct((grid * half, 2 * d), jnp.float32),
    )(emb_t)


def _sc_gather_mean(idx2d, pair_tab):
    sh_blk = VBK.bit_length() - 1
    sh_half = sh_blk - 1
    mask_half = (VBK // 2) - 1
    info = plsc.get_sparse_core_info()
    nc, ns = info.num_cores, info.num_subcores
    nw = nc * ns
    b_per_w = BATCH // nw
    rows_per_w = b_per_w * CONTEXT
    chunks_per_w = rows_per_w // IDX_CHUNK

    mesh = plsc.VectorSubcoreMesh(core_axis_name="c", subcore_axis_name="s")

    @functools.partial(
        pl.kernel,
        mesh=mesh,
        out_type=jax.ShapeDtypeStruct((BATCH, EMBED), jnp.float32),
        scratch_types=[
            pltpu.VMEM((chunks_per_w, IDX_CHUNK), jnp.int32),
            pltpu.VMEM((chunks_per_w, IDX_CHUNK), jnp.int32),
            pltpu.VMEM((rows_per_w, 2 * EMBED), jnp.float32),
            pltpu.VMEM((b_per_w, EMBED), jnp.float32),
            pltpu.SemaphoreType.DMA,
        ],
        compiler_params=pltpu.CompilerParams(use_tc_tiling_on_sc=False, needs_layout_passes=False),
    )
    def sc_kernel(idx_hbm, tab_hbm, out_hbm,
                  idx_v, off_v, rows_v, acc_v, sem):
        wid = lax.axis_index("s") * nc + lax.axis_index("c")
        base = wid * chunks_per_w
        pltpu.sync_copy(idx_hbm.at[pl.ds(base, chunks_per_w)], idx_v)
        for r in range(chunks_per_w):
            for c in range(0, IDX_CHUNK, 16):
                sl = pl.ds(c, 16)
                v = idx_v[r, sl]
                off_v[r, sl] = ((v >> sh_half) & 1) << 6
                idx_v[r, sl] = ((v >> sh_blk) << sh_half) | (v & mask_half)
        copies = []
        for j in range(chunks_per_w):
            copies.append(pltpu.async_copy(
                tab_hbm.at[idx_v.at[j]],
                rows_v.at[pl.ds(j * IDX_CHUNK, IDX_CHUNK)],
                sem))
        for c in copies:
            c.wait()

        inv = jnp.float32(1.0 / CONTEXT)
        lanes = lax.iota(jnp.int32, 16)

        def body(b, carry):
            row0 = b * CONTEXT
            accs = [jnp.zeros((16,), jnp.float32) for _ in range(4)]
            for t in range(CONTEXT):
                j = row0 + t
                jr = jnp.full((16,), j // IDX_CHUNK, jnp.int32)
                jc = jnp.full((16,), j % IDX_CHUNK, jnp.int32)
                off = plsc.load_gather(off_v, [jr, jc])
                jrow = jnp.full((16,), j, jnp.int32)
                for d in range(4):
                    col = off + (d * 16 + lanes)
                    accs[d] = accs[d] + plsc.load_gather(rows_v, [jrow, col])
            for d in range(4):
                acc_v[b, pl.ds(d * 16, 16)] = accs[d] * inv
            return carry

        lax.fori_loop(0, b_per_w, body, 0)
        pltpu.sync_copy(acc_v, out_hbm.at[pl.ds(wid * b_per_w, b_per_w)])

    return sc_kernel(idx2d, pair_tab)


def _tc_matmul_t(ctx, w_t, linear_b, vb=4096):
    v = w_t.shape[1]
    grid = pl.cdiv(v, vb)

    def mm_body(w_ref, x_ref, b_ref, o_ref):
        o_ref[...] = lax.dot_general(
            w_ref[...], x_ref[...],
            (((0,), (1,)), ((), ())),
            preferred_element_type=jnp.float32,
        ) + b_ref[...][:, None]

    return pl.pallas_call(
        mm_body,
        grid=(grid,),
        in_specs=[
            pl.BlockSpec((EMBED, vb), lambda j: (0, j)),
            pl.BlockSpec((BATCH, EMBED), lambda j: (0, 0)),
            pl.BlockSpec((vb,), lambda j: (j,)),
        ],
        out_specs=pl.BlockSpec((vb, BATCH), lambda j: (j, 0)),
        out_shape=jax.ShapeDtypeStruct((v, BATCH), jnp.float32),
    )(w_t, ctx, linear_b)


def kernel(context_words, embeddings, linear_w, linear_b):
    idx2d = context_words.astype(jnp.int32).reshape(-1, IDX_CHUNK)
    pair_tab = _tc_pair_table(embeddings.T)
    ctx = _sc_gather_mean(idx2d, pair_tab)
    out_t = _tc_matmul_t(ctx, linear_w.T, linear_b)
    return out_t.T
